# Initial kernel scaffold; baseline (speedup 1.0000x reference)
import jax
import jax.numpy as jnp
from jax.experimental import pallas as pl


def kernel(x, edge_index, batch, W1, b1, W2, b2, Wl1, bl1, Wl2, bl2):
    raise NotImplementedError("write your pallas kernel here")


# speedup vs baseline: 9.8594x; 9.8594x over previous
import jax
import jax.numpy as jnp
from jax import lax
from jax.experimental import pallas as pl
from jax.experimental.pallas import tpu as pltpu
from jax.experimental.pallas import tpu_sc as plsc

N = 10000
D = 128
H = 128
HF = H // 2
G = 256
E = 320000

NC = 2
NS = 16
NW = NC * NS

EB = 128
NB = 80
EP = NW * NB * EB

NPAD = 10240
BLK = 1024
NG = NPAD // BLK
RT = NPAD // NS

_HIGH = jax.lax.Precision.HIGHEST


def _dot(a, b):
    return jnp.dot(a, b, preferred_element_type=jnp.float32)


def _dot_exact(a, b):
    return jnp.dot(a, b, preferred_element_type=jnp.float32, precision=_HIGH)


def _sc_mesh():
    return plsc.VectorSubcoreMesh(
        core_axis_name="c", subcore_axis_name="s", num_cores=NC,
        num_subcores=NS)


def _deg_body(dst_hbm, out_hbm, dst_v, ones_v, stage_v, acc_sh):
    c = lax.axis_index("c")
    s = lax.axis_index("s")
    w = c * NS + s
    zeros16 = jnp.zeros((16,), jnp.float32)
    ones16 = jnp.ones((16,), jnp.float32)

    def _zrow(i, carry):
        stage_v[i, :] = zeros16
        return carry

    lax.fori_loop(0, RT, _zrow, 0)

    def _orow(i, carry):
        ones_v[i, :] = ones16
        return carry

    lax.fori_loop(0, EB, _orow, 0)

    pltpu.sync_copy(stage_v, acc_sh.at[pl.ds(s * RT, RT)])
    plsc.subcore_barrier()

    pltpu.sync_copy(dst_hbm.at[w], dst_v)

    def _blk(b, carry):
        pltpu.sync_copy(ones_v, acc_sh.at[dst_v.at[b]], add=True)
        return carry

    lax.fori_loop(0, NB, _blk, 0)
    plsc.subcore_barrier()

    pltpu.sync_copy(acc_sh.at[pl.ds(s * RT, RT)],
                    out_hbm.at[pl.ds(c * NPAD + s * RT, RT)])


def _deg_call(dstp):
    fn = pl.kernel(
        _deg_body,
        out_type=jax.ShapeDtypeStruct((NC * NPAD, 16), jnp.float32),
        mesh=_sc_mesh(),
        scratch_types=[
            pltpu.VMEM((NB, EB), jnp.int32),
            pltpu.VMEM((EB, 16), jnp.float32),
            pltpu.VMEM((RT, 16), jnp.float32),
            pltpu.VMEM_SHARED((NPAD, 16), jnp.float32),
        ],
        compiler_params=pltpu.CompilerParams(use_tc_tiling_on_sc=False),
    )
    return fn(dstp)


def _scatter_body(hsA, hsB, src_hbm, dst_hbm, outA, outB,
                  src_v, dst_v, buf0, buf1, zbuf, acc_sh, sem0, sem1):
    c = lax.axis_index("c")
    s = lax.axis_index("s")
    w = c * NS + s
    zeros16 = jnp.zeros((16,), jnp.float32)

    def _zrow(i, carry):
        for j in range(HF // 16):
            zbuf[i, pl.ds(j * 16, 16)] = zeros16
        return carry

    lax.fori_loop(0, EB, _zrow, 0)

    pltpu.sync_copy(src_hbm.at[w], src_v)
    pltpu.sync_copy(dst_hbm.at[w], dst_v)

    for hs_hbm, out_hbm in ((hsA, outA), (hsB, outB)):
        plsc.subcore_barrier()
        for k in range(RT // EB):
            pltpu.sync_copy(zbuf, acc_sh.at[pl.ds(s * RT + k * EB, EB)])
        plsc.subcore_barrier()

        pltpu.async_copy(hs_hbm.at[src_v.at[0]], buf0, sem0)

        def _pair(i, carry):
            b = i * 2
            pltpu.async_copy(hs_hbm.at[src_v.at[b + 1]], buf1, sem1)
            pltpu.make_async_copy(hs_hbm.at[src_v.at[b]], buf0, sem0).wait()
            pltpu.sync_copy(buf0, acc_sh.at[dst_v.at[b]], add=True)

            @pl.when(b + 2 < NB)
            def _():
                pltpu.async_copy(hs_hbm.at[src_v.at[b + 2]], buf0, sem0)

            pltpu.make_async_copy(hs_hbm.at[src_v.at[b + 1]], buf1,
                                  sem1).wait()
            pltpu.sync_copy(buf1, acc_sh.at[dst_v.at[b + 1]], add=True)
            return carry

        lax.fori_loop(0, NB // 2, _pair, 0)
        plsc.subcore_barrier()

        pltpu.sync_copy(acc_sh.at[pl.ds(s * RT, RT)],
                        out_hbm.at[pl.ds(c * NPAD + s * RT, RT)])


def _scatter_call(hsA, hsB, srcp, dstp):
    fn = pl.kernel(
        _scatter_body,
        out_type=[jax.ShapeDtypeStruct((NC * NPAD, HF), jnp.float32),
                  jax.ShapeDtypeStruct((NC * NPAD, HF), jnp.float32)],
        mesh=_sc_mesh(),
        scratch_types=[
            pltpu.VMEM((NB, EB), jnp.int32),
            pltpu.VMEM((NB, EB), jnp.int32),
            pltpu.VMEM((EB, HF), jnp.float32),
            pltpu.VMEM((EB, HF), jnp.float32),
            pltpu.VMEM((EB, HF), jnp.float32),
            pltpu.VMEM_SHARED((NPAD, HF), jnp.float32),
            pltpu.SemaphoreType.DMA,
            pltpu.SemaphoreType.DMA,
        ],
        compiler_params=pltpu.CompilerParams(use_tc_tiling_on_sc=False),
    )
    return fn(hsA, hsB, srcp, dstp)


def _mm1_body(x_ref, w_ref, deg0_ref, deg1_ref, hsA_ref, hsB_ref, dinv_ref):
    i = pl.program_id(0)
    deg = deg0_ref[:, 0:1] + deg1_ref[:, 0:1] + 1.0
    rows = i * BLK + lax.broadcasted_iota(jnp.int32, (BLK, 1), 0)
    dinv = jnp.where(rows < N, lax.rsqrt(deg), 0.0)
    hs = dinv * _dot(x_ref[...], w_ref[...])
    hsA_ref[...] = hs[:, :HF]
    hsB_ref[...] = hs[:, HF:]
    dinv_ref[...] = dinv


def _mm1_call(x_pad, w1, degp):
    return pl.pallas_call(
        _mm1_body,
        grid=(NG,),
        in_specs=[
            pl.BlockSpec((BLK, D), lambda i: (i, 0)),
            pl.BlockSpec((D, H), lambda i: (0, 0)),
            pl.BlockSpec((BLK, 16), lambda i: (i, 0)),
            pl.BlockSpec((BLK, 16), lambda i: (NG + i, 0)),
        ],
        out_specs=[
            pl.BlockSpec((BLK, HF), lambda i: (i, 0)),
            pl.BlockSpec((BLK, HF), lambda i: (i, 0)),
            pl.BlockSpec((BLK, 1), lambda i: (i, 0)),
        ],
        out_shape=[
            jax.ShapeDtypeStruct((NPAD, HF), jnp.float32),
            jax.ShapeDtypeStruct((NPAD, HF), jnp.float32),
            jax.ShapeDtypeStruct((NPAD, 1), jnp.float32),
        ],
    )(x_pad, w1, degp, degp)


def _agg(pA0, pA1, pB0, pB1, hsA, hsB):
    return jnp.concatenate([pA0 + pA1 + hsA, pB0 + pB1 + hsB], axis=1)


def _mm2_body(pA0, pA1, pB0, pB1, hsA, hsB, dinv_ref, b1_ref, w2_ref,
              hs2A_ref, hs2B_ref):
    dinv = dinv_ref[...]
    agg = _agg(pA0[...], pA1[...], pB0[...], pB1[...], hsA[...], hsB[...])
    l1 = jnp.maximum(dinv * agg + b1_ref[...], 0.0)
    hs2 = dinv * _dot(l1, w2_ref[...])
    hs2A_ref[...] = hs2[:, :HF]
    hs2B_ref[...] = hs2[:, HF:]


def _part_specs():
    return [
        pl.BlockSpec((BLK, HF), lambda i: (i, 0)),
        pl.BlockSpec((BLK, HF), lambda i: (NG + i, 0)),
    ]


def _mm2_call(pA, pB, hsA, hsB, dinv, b1, w2):
    return pl.pallas_call(
        _mm2_body,
        grid=(NG,),
        in_specs=(
            _part_specs() + _part_specs() + [
                pl.BlockSpec((BLK, HF), lambda i: (i, 0)),
                pl.BlockSpec((BLK, HF), lambda i: (i, 0)),
                pl.BlockSpec((BLK, 1), lambda i: (i, 0)),
                pl.BlockSpec((1, H), lambda i: (0, 0)),
                pl.BlockSpec((H, H), lambda i: (0, 0)),
            ]),
        out_specs=[
            pl.BlockSpec((BLK, HF), lambda i: (i, 0)),
            pl.BlockSpec((BLK, HF), lambda i: (i, 0)),
        ],
        out_shape=[
            jax.ShapeDtypeStruct((NPAD, HF), jnp.float32),
            jax.ShapeDtypeStruct((NPAD, HF), jnp.float32),
        ],
    )(pA, pA, pB, pB, hsA, hsB, dinv, b1, w2)


def _pool_body(qA0, qA1, qB0, qB1, hsA, hsB, dinv_ref, b2_ref, batch_ref,
               wl1_ref, bl1_ref, wl2_ref, bl2_ref, out_ref, sums, counts):
    i = pl.program_id(0)

    @pl.when(i == 0)
    def _():
        sums[...] = jnp.zeros_like(sums)
        counts[...] = jnp.zeros_like(counts)

    dinv = dinv_ref[...]
    agg = _agg(qA0[...], qA1[...], qB0[...], qB1[...], hsA[...], hsB[...])
    l2 = jnp.maximum(dinv * agg + b2_ref[...], 0.0)
    bi = batch_ref[...]
    oh = (lax.broadcasted_iota(jnp.int32, (G, BLK), 0)
          == jnp.broadcast_to(bi, (G, BLK))).astype(jnp.float32)
    sums[...] += _dot_exact(oh, l2)
    counts[...] += jnp.sum(oh, axis=1, keepdims=True)

    @pl.when(i == NG - 1)
    def _():
        pooled = sums[...] / jnp.maximum(counts[...], 1.0)
        a = jnp.maximum(_dot(pooled, wl1_ref[...]) + bl1_ref[...], 0.0)
        out_ref[...] = _dot(a, wl2_ref[...]) + bl2_ref[...]


def _pool_call(qA, qB, hsA, hsB, dinv, b2, batch2d, wl1, bl1, wl2, bl2):
    return pl.pallas_call(
        _pool_body,
        grid=(NG,),
        in_specs=(
            _part_specs() + _part_specs() + [
                pl.BlockSpec((BLK, HF), lambda i: (i, 0)),
                pl.BlockSpec((BLK, HF), lambda i: (i, 0)),
                pl.BlockSpec((BLK, 1), lambda i: (i, 0)),
                pl.BlockSpec((1, H), lambda i: (0, 0)),
                pl.BlockSpec((1, BLK), lambda i: (0, i)),
                pl.BlockSpec((H, H // 2), lambda i: (0, 0)),
                pl.BlockSpec((1, H // 2), lambda i: (0, 0)),
                pl.BlockSpec((H // 2, 1), lambda i: (0, 0)),
                pl.BlockSpec((1, 1), lambda i: (0, 0)),
            ]),
        out_specs=pl.BlockSpec((G, 1), lambda i: (0, 0)),
        out_shape=jax.ShapeDtypeStruct((G, 1), jnp.float32),
        scratch_shapes=[
            pltpu.VMEM((G, H), jnp.float32),
            pltpu.VMEM((G, 1), jnp.float32),
        ],
    )(qA, qA, qB, qB, hsA, hsB, dinv, b2, batch2d, wl1, bl1, wl2, bl2)


def kernel(x, edge_index, batch, W1, b1, W2, b2, Wl1, bl1, Wl2, bl2):
    src = edge_index[0]
    dst = edge_index[1]
    srcp = jnp.concatenate(
        [src, jnp.zeros((EP - E,), jnp.int32)]).reshape(NW, NB, EB)
    dstp = jnp.concatenate(
        [dst, jnp.full((EP - E,), N, jnp.int32)]).reshape(NW, NB, EB)
    x_pad = jnp.pad(x, ((0, NPAD - N), (0, 0)))
    batch2d = jnp.pad(batch, (0, NPAD - N),
                      constant_values=G).reshape(1, NPAD)

    degp = _deg_call(dstp)
    hsA, hsB, dinv = _mm1_call(x_pad, W1, degp)
    pA, pB = _scatter_call(hsA, hsB, srcp, dstp)
    hs2A, hs2B = _mm2_call(pA, pB, hsA, hsB, dinv, b1.reshape(1, H), W2)
    qA, qB = _scatter_call(hs2A, hs2B, srcp, dstp)
    return _pool_call(qA, qB, hs2A, hs2B, dinv, b2.reshape(1, H), batch2d,
                      Wl1, bl1.reshape(1, H // 2), Wl2, bl2.reshape(1, 1))


# speedup vs baseline: 15.5222x; 1.5744x over previous
import jax
import jax.numpy as jnp
from jax import lax
from jax.experimental import pallas as pl
from jax.experimental.pallas import tpu as pltpu
from jax.experimental.pallas import tpu_sc as plsc

N = 10000
D = 128
H = 128
G = 256
E = 320000

NC = 2
NS = 16
NW = NC * NS

EB = 128
NB = 80
EP = NW * NB * EB

NPAD = 10240
BLK = 1024
NG = NPAD // BLK
RT = NPAD // NS

_HIGH = jax.lax.Precision.HIGHEST


def _dot(a, b):
    return jnp.dot(a, b, preferred_element_type=jnp.float32)


def _dot_exact(a, b):
    return jnp.dot(a, b, preferred_element_type=jnp.float32, precision=_HIGH)


def _sc_mesh():
    return plsc.VectorSubcoreMesh(
        core_axis_name="c", subcore_axis_name="s", num_cores=NC,
        num_subcores=NS)


def _deg_body(dst_hbm, out_hbm, dst_v, ones_v, stage_v, acc_sh):
    c = lax.axis_index("c")
    s = lax.axis_index("s")
    w = c * NS + s
    zeros16 = jnp.zeros((16,), jnp.float32)
    ones16 = jnp.ones((16,), jnp.float32)

    def _zrow(i, carry):
        stage_v[i, :] = zeros16
        return carry

    lax.fori_loop(0, RT, _zrow, 0)

    def _orow(i, carry):
        ones_v[i, :] = ones16
        return carry

    lax.fori_loop(0, EB, _orow, 0)

    pltpu.sync_copy(stage_v, acc_sh.at[pl.ds(s * RT, RT)])
    plsc.subcore_barrier()

    pltpu.sync_copy(dst_hbm.at[w], dst_v)

    def _blk(b, carry):
        pltpu.sync_copy(ones_v, acc_sh.at[dst_v.at[b]], add=True)
        return carry

    lax.fori_loop(0, NB, _blk, 0)
    plsc.subcore_barrier()

    pltpu.sync_copy(acc_sh.at[pl.ds(s * RT, RT)],
                    out_hbm.at[pl.ds(c * NPAD + s * RT, RT)])


def _deg_call(dstp):
    fn = pl.kernel(
        _deg_body,
        out_type=jax.ShapeDtypeStruct((NC * NPAD, 16), jnp.float32),
        mesh=_sc_mesh(),
        scratch_types=[
            pltpu.VMEM((NB, EB), jnp.int32),
            pltpu.VMEM((EB, 16), jnp.float32),
            pltpu.VMEM((RT, 16), jnp.float32),
            pltpu.VMEM_SHARED((NPAD, 16), jnp.float32),
        ],
        compiler_params=pltpu.CompilerParams(use_tc_tiling_on_sc=False),
    )
    return fn(dstp)


def _scatter_body(hs_hbm, src_hbm, dst_hbm, out_hbm,
                  src_v, dst_v, buf0, buf1, zbuf, acc_sh, sem0, sem1):
    c = lax.axis_index("c")
    s = lax.axis_index("s")
    w = c * NS + s
    zeros32 = jnp.zeros((32,), jnp.bfloat16)

    def _zrow(i, carry):
        for j in range(H // 32):
            zbuf[i, pl.ds(j * 32, 32)] = zeros32
        return carry

    lax.fori_loop(0, EB, _zrow, 0)

    pltpu.sync_copy(src_hbm.at[w], src_v)
    pltpu.sync_copy(dst_hbm.at[w], dst_v)

    for k in range(RT // EB):
        pltpu.sync_copy(zbuf, acc_sh.at[pl.ds(s * RT + k * EB, EB)])
    plsc.subcore_barrier()

    pltpu.async_copy(hs_hbm.at[src_v.at[0]], buf0, sem0)

    def _pair(i, carry):
        b = i * 2
        pltpu.async_copy(hs_hbm.at[src_v.at[b + 1]], buf1, sem1)
        pltpu.make_async_copy(hs_hbm.at[src_v.at[b]], buf0, sem0).wait()
        pltpu.sync_copy(buf0, acc_sh.at[dst_v.at[b]], add=True)

        @pl.when(b + 2 < NB)
        def _():
            pltpu.async_copy(hs_hbm.at[src_v.at[b + 2]], buf0, sem0)

        pltpu.make_async_copy(hs_hbm.at[src_v.at[b + 1]], buf1,
                              sem1).wait()
        pltpu.sync_copy(buf1, acc_sh.at[dst_v.at[b + 1]], add=True)
        return carry

    lax.fori_loop(0, NB // 2, _pair, 0)
    plsc.subcore_barrier()

    pltpu.sync_copy(acc_sh.at[pl.ds(s * RT, RT)],
                    out_hbm.at[pl.ds(c * NPAD + s * RT, RT)])


def _scatter_call(hsb, srcp, dstp):
    fn = pl.kernel(
        _scatter_body,
        out_type=jax.ShapeDtypeStruct((NC * NPAD, H), jnp.bfloat16),
        mesh=_sc_mesh(),
        scratch_types=[
            pltpu.VMEM((NB, EB), jnp.int32),
            pltpu.VMEM((NB, EB), jnp.int32),
            pltpu.VMEM((EB, H), jnp.bfloat16),
            pltpu.VMEM((EB, H), jnp.bfloat16),
            pltpu.VMEM((EB, H), jnp.bfloat16),
            pltpu.VMEM_SHARED((NPAD, H), jnp.bfloat16),
            pltpu.SemaphoreType.DMA,
            pltpu.SemaphoreType.DMA,
        ],
        compiler_params=pltpu.CompilerParams(use_tc_tiling_on_sc=False),
    )
    return fn(hsb, srcp, dstp)


def _mm1_body(x_ref, w_ref, deg0_ref, deg1_ref, hs_ref, hsb_ref, dinv_ref):
    i = pl.program_id(0)
    deg = deg0_ref[:, 0:1] + deg1_ref[:, 0:1] + 1.0
    rows = i * BLK + lax.broadcasted_iota(jnp.int32, (BLK, 1), 0)
    dinv = jnp.where(rows < N, lax.rsqrt(deg), 0.0)
    hs = dinv * _dot(x_ref[...], w_ref[...])
    hs_ref[...] = hs
    hsb_ref[...] = hs.astype(jnp.bfloat16)
    dinv_ref[...] = dinv


def _mm1_call(x_pad, w1, degp):
    return pl.pallas_call(
        _mm1_body,
        grid=(NG,),
        in_specs=[
            pl.BlockSpec((BLK, D), lambda i: (i, 0)),
            pl.BlockSpec((D, H), lambda i: (0, 0)),
            pl.BlockSpec((BLK, 16), lambda i: (i, 0)),
            pl.BlockSpec((BLK, 16), lambda i: (NG + i, 0)),
        ],
        out_specs=[
            pl.BlockSpec((BLK, H), lambda i: (i, 0)),
            pl.BlockSpec((BLK, H), lambda i: (i, 0)),
            pl.BlockSpec((BLK, 1), lambda i: (i, 0)),
        ],
        out_shape=[
            jax.ShapeDtypeStruct((NPAD, H), jnp.float32),
            jax.ShapeDtypeStruct((NPAD, H), jnp.bfloat16),
            jax.ShapeDtypeStruct((NPAD, 1), jnp.float32),
        ],
    )(x_pad, w1, degp, degp)


def _mm2_body(p0, p1, hs_ref, dinv_ref, b1_ref, w2_ref, hs2_ref, hs2b_ref):
    dinv = dinv_ref[...]
    agg = (p0[...].astype(jnp.float32) + p1[...].astype(jnp.float32)
           + hs_ref[...])
    l1 = jnp.maximum(dinv * agg + b1_ref[...], 0.0)
    hs2 = dinv * _dot(l1, w2_ref[...])
    hs2_ref[...] = hs2
    hs2b_ref[...] = hs2.astype(jnp.bfloat16)


def _part_specs():
    return [
        pl.BlockSpec((BLK, H), lambda i: (i, 0)),
        pl.BlockSpec((BLK, H), lambda i: (NG + i, 0)),
    ]


def _mm2_call(p, hs1, dinv, b1, w2):
    return pl.pallas_call(
        _mm2_body,
        grid=(NG,),
        in_specs=(
            _part_specs() + [
                pl.BlockSpec((BLK, H), lambda i: (i, 0)),
                pl.BlockSpec((BLK, 1), lambda i: (i, 0)),
                pl.BlockSpec((1, H), lambda i: (0, 0)),
                pl.BlockSpec((H, H), lambda i: (0, 0)),
            ]),
        out_specs=[
            pl.BlockSpec((BLK, H), lambda i: (i, 0)),
            pl.BlockSpec((BLK, H), lambda i: (i, 0)),
        ],
        out_shape=[
            jax.ShapeDtypeStruct((NPAD, H), jnp.float32),
            jax.ShapeDtypeStruct((NPAD, H), jnp.bfloat16),
        ],
    )(p, p, hs1, dinv, b1, w2)


def _pool_body(q0, q1, hs_ref, dinv_ref, b2_ref, batch_ref,
               wl1_ref, bl1_ref, wl2_ref, bl2_ref, out_ref, sums, counts):
    i = pl.program_id(0)

    @pl.when(i == 0)
    def _():
        sums[...] = jnp.zeros_like(sums)
        counts[...] = jnp.zeros_like(counts)

    dinv = dinv_ref[...]
    agg = (q0[...].astype(jnp.float32) + q1[...].astype(jnp.float32)
           + hs_ref[...])
    l2 = jnp.maximum(dinv * agg + b2_ref[...], 0.0)
    bi = batch_ref[...]
    oh = (lax.broadcasted_iota(jnp.int32, (G, BLK), 0)
          == jnp.broadcast_to(bi, (G, BLK))).astype(jnp.float32)
    sums[...] += _dot_exact(oh, l2)
    counts[...] += jnp.sum(oh, axis=1, keepdims=True)

    @pl.when(i == NG - 1)
    def _():
        pooled = sums[...] / jnp.maximum(counts[...], 1.0)
        a = jnp.maximum(_dot(pooled, wl1_ref[...]) + bl1_ref[...], 0.0)
        out_ref[...] = _dot(a, wl2_ref[...]) + bl2_ref[...]


def _pool_call(q, hs2, dinv, b2, batch2d, wl1, bl1, wl2, bl2):
    return pl.pallas_call(
        _pool_body,
        grid=(NG,),
        in_specs=(
            _part_specs() + [
                pl.BlockSpec((BLK, H), lambda i: (i, 0)),
                pl.BlockSpec((BLK, 1), lambda i: (i, 0)),
                pl.BlockSpec((1, H), lambda i: (0, 0)),
                pl.BlockSpec((1, BLK), lambda i: (0, i)),
                pl.BlockSpec((H, H // 2), lambda i: (0, 0)),
                pl.BlockSpec((1, H // 2), lambda i: (0, 0)),
                pl.BlockSpec((H // 2, 1), lambda i: (0, 0)),
                pl.BlockSpec((1, 1), lambda i: (0, 0)),
            ]),
        out_specs=pl.BlockSpec((G, 1), lambda i: (0, 0)),
        out_shape=jax.ShapeDtypeStruct((G, 1), jnp.float32),
        scratch_shapes=[
            pltpu.VMEM((G, H), jnp.float32),
            pltpu.VMEM((G, 1), jnp.float32),
        ],
    )(q, q, hs2, dinv, b2, batch2d, wl1, bl1, wl2, bl2)


def kernel(x, edge_index, batch, W1, b1, W2, b2, Wl1, bl1, Wl2, bl2):
    src = edge_index[0]
    dst = edge_index[1]
    srcp = jnp.concatenate(
        [src, jnp.zeros((EP - E,), jnp.int32)]).reshape(NW, NB, EB)
    dstp = jnp.concatenate(
        [dst, jnp.full((EP - E,), N, jnp.int32)]).reshape(NW, NB, EB)
    x_pad = jnp.pad(x, ((0, NPAD - N), (0, 0)))
    batch2d = jnp.pad(batch, (0, NPAD - N),
                      constant_values=G).reshape(1, NPAD)

    degp = _deg_call(dstp)
    hs1, hs1b, dinv = _mm1_call(x_pad, W1, degp)
    p = _scatter_call(hs1b, srcp, dstp)
    hs2, hs2b = _mm2_call(p, hs1, dinv, b1.reshape(1, H), W2)
    q = _scatter_call(hs2b, srcp, dstp)
    return _pool_call(q, hs2, dinv, b2.reshape(1, H), batch2d,
                      Wl1, bl1.reshape(1, H // 2), Wl2, bl2.reshape(1, 1))
